# Initial kernel scaffold; baseline (speedup 1.0000x reference)
import jax
import jax.numpy as jnp
from jax.experimental import pallas as pl


def kernel(position, velocity, init, W1, b1, W2, b2, W3, b3):
    raise NotImplementedError("write your pallas kernel here")


# speedup vs baseline: 22.9803x; 22.9803x over previous
import functools

import numpy as np
import jax
import jax.numpy as jnp
from jax import lax
from jax.experimental import pallas as pl
from jax.experimental.pallas import tpu as pltpu
from jax.experimental.pallas import tpu_sc as plsc

_B, _N, _K = 4, 4096, 10
_S = _K + 1
_R = 256
_RC = 1024

_NC, _NS = 2, 16
_NW = _NC * _NS
_TOT = _B * _N * _S
_PW = _TOT // _NW
_CH = 128
_NT = _PW // _CH
_GRP = 11


def _topk_body(pos_ref, posT_ref, idx_ref):
    b = pl.program_id(0)
    pos_r = pos_ref[0]
    posT = posT_ref[0]
    p2r = jnp.sum(pos_r * pos_r, axis=1, keepdims=True)
    p2c = jnp.sum(posT * posT, axis=0, keepdims=True)
    cross = jnp.dot(pos_r, posT, preferred_element_type=jnp.float32)
    d2 = p2r + p2c - 2.0 * cross
    col = lax.broadcasted_iota(jnp.int32, d2.shape, 1)
    slots = []
    for _ in range(_S):
        m = jnp.min(d2, axis=1, keepdims=True)
        idx_t = jnp.min(jnp.where(d2 == m, col, _N), axis=1, keepdims=True)
        slots.append(idx_t)
        d2 = jnp.where(col == idx_t, jnp.inf, d2)
    idx = jnp.concatenate(slots, axis=1)
    idx_ref[0] = idx + b * _N


def _topk_call(position, posT):
    return pl.pallas_call(
        _topk_body,
        grid=(_B, _N // _R),
        in_specs=[
            pl.BlockSpec((1, _R, 3), lambda b, r: (b, r, 0)),
            pl.BlockSpec((1, 3, _N), lambda b, r: (b, 0, 0)),
        ],
        out_specs=pl.BlockSpec((1, _R, _S), lambda b, r: (b, r, 0)),
        out_shape=jax.ShapeDtypeStruct((_B, _N, _S), jnp.int32),
    )(position, posT)


def _gather_body(table_ref, idx_ref, out_ref, idx_v, rows_v, sem):
    wid = lax.axis_index("s") * _NC + lax.axis_index("c")
    pltpu.sync_copy(idx_ref.at[wid], idx_v)

    def grp(gi, carry):
        handles = []
        for j in range(_GRP):
            tr = gi * _GRP + j
            cp = pltpu.async_copy(
                table_ref.at[idx_v.at[tr]],
                rows_v.at[pl.ds(tr * _CH, _CH)],
                sem,
            )
            handles.append(cp)
        for cp in handles:
            cp.wait()
        return carry

    lax.fori_loop(0, _NT // _GRP, grp, 0)
    pltpu.sync_copy(rows_v, out_ref.at[wid])


@functools.cache
def _gather_call():
    return functools.partial(
        pl.kernel,
        mesh=plsc.VectorSubcoreMesh(core_axis_name="c", subcore_axis_name="s",
                                    num_cores=_NC, num_subcores=_NS),
        out_type=jax.ShapeDtypeStruct((_NW, _PW, 8), jnp.float32),
        scratch_types=[
            pltpu.VMEM((_NT, _CH), jnp.int32),
            pltpu.VMEM((_PW, 8), jnp.float32),
            pltpu.SemaphoreType.DMA,
        ],
        compiler_params=pltpu.CompilerParams(use_tc_tiling_on_sc=False),
    )(_gather_body)


def _placement_matrices():
    P = np.zeros((8 * _S, 66), dtype=np.float32)
    for d in range(3):
        for t in range(1, _S):
            P[8 * t + d, 3 * (t - 1) + d] = 1.0
        for t in range(_S):
            P[8 * t + 3 + d, 30 + 3 * t + d] = 1.0
    R = np.zeros((3, 66), dtype=np.float32)
    Q3 = np.zeros((3, 6), dtype=np.float32)
    for d in range(3):
        for t in range(_K):
            R[d, 3 * t + d] = -1.0
        Q3[d, d] = 1.0
    return jnp.asarray(P), jnp.asarray(R), jnp.asarray(Q3)


def _mlp_body(g_ref, pos_ref, init_ref, W1_ref, W2_ref, W3_ref, b1_ref,
              b2_ref, b3_ref, P_ref, R_ref, Q3_ref, out_ref):
    f32 = jnp.float32
    hi = lax.Precision.HIGHEST
    W32 = jnp.dot(W3_ref[...], W2_ref[...], precision=hi,
                  preferred_element_type=f32)
    Weff = jnp.dot(W32, W1_ref[...], precision=hi,
                   preferred_element_type=f32)
    A = lax.dot_general(P_ref[...], Weff, (((1,), (1,)), ((), ())),
                        precision=hi, preferred_element_type=f32)
    C = lax.dot_general(R_ref[...], Weff, (((1,), (1,)), ((), ())),
                        precision=hi,
                        preferred_element_type=f32) + Q3_ref[...]
    beff = (lax.dot_general(b2_ref[...], W3_ref[...], (((1,), (1,)), ((), ())),
                            precision=hi, preferred_element_type=f32)
            + lax.dot_general(b1_ref[...], W32, (((1,), (1,)), ((), ())),
                              precision=hi, preferred_element_type=f32)
            + b3_ref[...])
    db = lax.dot_general(init_ref[0], Weff[:, 63:66],
                         (((1,), (1,)), ((), ())),
                         precision=hi, preferred_element_type=f32) + beff
    out_ref[0] = (jnp.dot(g_ref[0], A, precision=hi,
                          preferred_element_type=f32)
                  + jnp.dot(pos_ref[0], C, precision=hi,
                            preferred_element_type=f32) + db)


def _mlp_call(G, position, init3, W1, W2, W3, b1r, b2r, b3r, P, R, Q3):
    full = lambda shape: pl.BlockSpec(shape, lambda b, r: tuple(0 for _ in shape))
    return pl.pallas_call(
        _mlp_body,
        grid=(_B, _N // _RC),
        in_specs=[
            pl.BlockSpec((1, _RC, 8 * _S), lambda b, r: (b, r, 0)),
            pl.BlockSpec((1, _RC, 3), lambda b, r: (b, r, 0)),
            pl.BlockSpec((1, 1, 3), lambda b, r: (b, 0, 0)),
            full(W1.shape), full(W2.shape), full(W3.shape),
            full(b1r.shape), full(b2r.shape), full(b3r.shape),
            full(P.shape), full(R.shape), full(Q3.shape),
        ],
        out_specs=pl.BlockSpec((1, _RC, 6), lambda b, r: (b, r, 0)),
        out_shape=jax.ShapeDtypeStruct((_B, _N, 6), jnp.float32),
    )(G, position, init3, W1, W2, W3, b1r, b2r, b3r, P, R, Q3)


@jax.jit
def kernel(position, velocity, init, W1, b1, W2, b2, W3, b3):
    posT = jnp.swapaxes(position, 1, 2)
    idx = _topk_call(position, posT)
    idx3 = idx.reshape(_NW, _NT, _CH)

    table = jnp.concatenate(
        [position, velocity, jnp.zeros((_B, _N, 2), jnp.float32)], axis=-1
    ).reshape(_B * _N, 8)
    G = _gather_call()(table, idx3).reshape(_B, _N, 8 * _S)

    P, R, Q3 = _placement_matrices()
    return _mlp_call(G, position, init.reshape(_B, 1, 3), W1, W2, W3,
                     b1.reshape(1, -1), b2.reshape(1, -1), b3.reshape(1, -1),
                     P, R, Q3)


# speedup vs baseline: 26.0005x; 1.1314x over previous
import functools

import numpy as np
import jax
import jax.numpy as jnp
from jax import lax
from jax.experimental import pallas as pl
from jax.experimental.pallas import tpu as pltpu
from jax.experimental.pallas import tpu_sc as plsc

_B, _N, _K = 4, 4096, 10
_S = _K + 1
_R = 256
_RC = 1024

_NC, _NS = 2, 16
_NW = _NC * _NS
_TOT = _B * _N * _S
_PW = _TOT // _NW
_CH = 128
_NT = _PW // _CH
_GRP = 11


def _topk_body(pos_ref, posT_ref, idx_ref):
    b = pl.program_id(0)
    pos_r = pos_ref[0]
    posT = posT_ref[0]
    p2r = jnp.sum(pos_r * pos_r, axis=1, keepdims=True)
    p2c = jnp.sum(posT * posT, axis=0, keepdims=True)
    cross = jnp.dot(pos_r, posT, preferred_element_type=jnp.float32)
    d2 = p2r + p2c - 2.0 * cross
    col = lax.broadcasted_iota(jnp.int32, d2.shape, 1)
    slots = []
    for _ in range(_S):
        m = jnp.min(d2, axis=1, keepdims=True)
        eq = d2 == m
        idx_t = jnp.min(jnp.where(eq, col, _N), axis=1, keepdims=True)
        slots.append(idx_t)
        d2 = jnp.where(eq, jnp.inf, d2)
    idx = jnp.concatenate(slots, axis=1)
    idx_ref[0] = idx + b * _N


def _topk_call(position, posT):
    return pl.pallas_call(
        _topk_body,
        grid=(_B, _N // _R),
        in_specs=[
            pl.BlockSpec((1, _R, 3), lambda b, r: (b, r, 0)),
            pl.BlockSpec((1, 3, _N), lambda b, r: (b, 0, 0)),
        ],
        out_specs=pl.BlockSpec((1, _R, _S), lambda b, r: (b, r, 0)),
        out_shape=jax.ShapeDtypeStruct((_B, _N, _S), jnp.int32),
    )(position, posT)


def _gather_body(table_ref, idx_ref, out_ref, idx_v, rows_v, sem):
    wid = lax.axis_index("s") * _NC + lax.axis_index("c")
    pltpu.sync_copy(idx_ref.at[wid], idx_v)

    def grp(gi, carry):
        handles = []
        for j in range(_GRP):
            tr = gi * _GRP + j
            cp = pltpu.async_copy(
                table_ref.at[idx_v.at[tr]],
                rows_v.at[pl.ds(tr * _CH, _CH)],
                sem,
            )
            handles.append(cp)
        for cp in handles:
            cp.wait()
        return carry

    lax.fori_loop(0, _NT // _GRP, grp, 0)
    pltpu.sync_copy(rows_v, out_ref.at[wid])


@functools.cache
def _gather_call():
    return functools.partial(
        pl.kernel,
        mesh=plsc.VectorSubcoreMesh(core_axis_name="c", subcore_axis_name="s",
                                    num_cores=_NC, num_subcores=_NS),
        out_type=jax.ShapeDtypeStruct((_NW, _PW, 8), jnp.float32),
        scratch_types=[
            pltpu.VMEM((_NT, _CH), jnp.int32),
            pltpu.VMEM((_PW, 8), jnp.float32),
            pltpu.SemaphoreType.DMA,
        ],
        compiler_params=pltpu.CompilerParams(use_tc_tiling_on_sc=False),
    )(_gather_body)


def _placement_matrices():
    P = np.zeros((8 * _S, 66), dtype=np.float32)
    for d in range(3):
        for t in range(1, _S):
            P[8 * t + d, 3 * (t - 1) + d] = 1.0
        for t in range(_S):
            P[8 * t + 3 + d, 30 + 3 * t + d] = 1.0
    R = np.zeros((3, 66), dtype=np.float32)
    Q3 = np.zeros((3, 6), dtype=np.float32)
    for d in range(3):
        for t in range(_K):
            R[d, 3 * t + d] = -1.0
        Q3[d, d] = 1.0
    return jnp.asarray(P), jnp.asarray(R), jnp.asarray(Q3)


def _mlp_body(g_ref, pos_ref, init_ref, W1_ref, W2_ref, W3_ref, b1_ref,
              b2_ref, b3_ref, P_ref, R_ref, Q3_ref, out_ref):
    f32 = jnp.float32
    hi = lax.Precision.HIGHEST
    W32 = jnp.dot(W3_ref[...], W2_ref[...], precision=hi,
                  preferred_element_type=f32)
    Weff = jnp.dot(W32, W1_ref[...], precision=hi,
                   preferred_element_type=f32)
    A = lax.dot_general(P_ref[...], Weff, (((1,), (1,)), ((), ())),
                        precision=hi, preferred_element_type=f32)
    C = lax.dot_general(R_ref[...], Weff, (((1,), (1,)), ((), ())),
                        precision=hi,
                        preferred_element_type=f32) + Q3_ref[...]
    beff = (lax.dot_general(b2_ref[...], W3_ref[...], (((1,), (1,)), ((), ())),
                            precision=hi, preferred_element_type=f32)
            + lax.dot_general(b1_ref[...], W32, (((1,), (1,)), ((), ())),
                              precision=hi, preferred_element_type=f32)
            + b3_ref[...])
    db = lax.dot_general(init_ref[0], Weff[:, 63:66],
                         (((1,), (1,)), ((), ())),
                         precision=hi, preferred_element_type=f32) + beff
    out_ref[0] = (jnp.dot(g_ref[0], A, precision=hi,
                          preferred_element_type=f32)
                  + jnp.dot(pos_ref[0], C, precision=hi,
                            preferred_element_type=f32) + db)


def _mlp_call(G, position, init3, W1, W2, W3, b1r, b2r, b3r, P, R, Q3):
    full = lambda shape: pl.BlockSpec(shape, lambda b, r: tuple(0 for _ in shape))
    return pl.pallas_call(
        _mlp_body,
        grid=(_B, _N // _RC),
        in_specs=[
            pl.BlockSpec((1, _RC, 8 * _S), lambda b, r: (b, r, 0)),
            pl.BlockSpec((1, _RC, 3), lambda b, r: (b, r, 0)),
            pl.BlockSpec((1, 1, 3), lambda b, r: (b, 0, 0)),
            full(W1.shape), full(W2.shape), full(W3.shape),
            full(b1r.shape), full(b2r.shape), full(b3r.shape),
            full(P.shape), full(R.shape), full(Q3.shape),
        ],
        out_specs=pl.BlockSpec((1, _RC, 6), lambda b, r: (b, r, 0)),
        out_shape=jax.ShapeDtypeStruct((_B, _N, 6), jnp.float32),
    )(G, position, init3, W1, W2, W3, b1r, b2r, b3r, P, R, Q3)


@jax.jit
def kernel(position, velocity, init, W1, b1, W2, b2, W3, b3):
    posT = jnp.swapaxes(position, 1, 2)
    idx = _topk_call(position, posT)
    idx3 = idx.reshape(_NW, _NT, _CH)

    table = jnp.concatenate(
        [position, velocity, jnp.zeros((_B, _N, 2), jnp.float32)], axis=-1
    ).reshape(_B * _N, 8)
    G = _gather_call()(table, idx3).reshape(_B, _N, 8 * _S)

    P, R, Q3 = _placement_matrices()
    return _mlp_call(G, position, init.reshape(_B, 1, 3), W1, W2, W3,
                     b1.reshape(1, -1), b2.reshape(1, -1), b3.reshape(1, -1),
                     P, R, Q3)


# speedup vs baseline: 28.3019x; 1.0885x over previous
import functools

import numpy as np
import jax
import jax.numpy as jnp
from jax import lax
from jax.experimental import pallas as pl
from jax.experimental.pallas import tpu as pltpu
from jax.experimental.pallas import tpu_sc as plsc

_B, _N, _K = 4, 4096, 10
_S = _K + 1
_R = 256
_RC = 1024

_NC, _NS = 2, 16
_NW = _NC * _NS
_TOT = _B * _N * _S
_PW = _TOT // _NW
_CH = 128
_NT = _PW // _CH
_GRP = 11


def _topk_body(pos_ref, posT_ref, idx_ref):
    b = pl.program_id(0)
    pos_r = pos_ref[0]
    posT = posT_ref[0]
    p2r = jnp.sum(pos_r * pos_r, axis=1, keepdims=True)
    p2c = jnp.sum(posT * posT, axis=0, keepdims=True)
    cross = jnp.dot(pos_r, posT, preferred_element_type=jnp.float32)
    d2 = p2r + p2c - 2.0 * cross
    colf = lax.broadcasted_iota(jnp.int32, d2.shape, 1).astype(jnp.float32)
    bigf = jnp.float32(_N)
    slots = []
    for _ in range(_S):
        m = jnp.min(d2, axis=1, keepdims=True)
        eq = d2 == m
        idx_t = jnp.min(jnp.where(eq, colf, bigf), axis=1, keepdims=True)
        slots.append(idx_t.astype(jnp.int32))
        d2 = jnp.where(eq, jnp.inf, d2)
    idx = jnp.concatenate(slots, axis=1)
    idx_ref[0] = idx + b * _N


def _topk_call(position, posT):
    return pl.pallas_call(
        _topk_body,
        grid=(_B, _N // _R),
        in_specs=[
            pl.BlockSpec((1, _R, 3), lambda b, r: (b, r, 0)),
            pl.BlockSpec((1, 3, _N), lambda b, r: (b, 0, 0)),
        ],
        out_specs=pl.BlockSpec((1, _R, _S), lambda b, r: (b, r, 0)),
        out_shape=jax.ShapeDtypeStruct((_B, _N, _S), jnp.int32),
    )(position, posT)


def _gather_body(table_ref, idx_ref, out_ref, idx_v, rows_v, sem):
    wid = lax.axis_index("s") * _NC + lax.axis_index("c")
    pltpu.sync_copy(idx_ref.at[wid], idx_v)

    def grp(gi, carry):
        handles = []
        for j in range(_GRP):
            tr = gi * _GRP + j
            cp = pltpu.async_copy(
                table_ref.at[idx_v.at[tr]],
                rows_v.at[pl.ds(tr * _CH, _CH)],
                sem,
            )
            handles.append(cp)
        for cp in handles:
            cp.wait()
        return carry

    lax.fori_loop(0, _NT // _GRP, grp, 0)
    pltpu.sync_copy(rows_v, out_ref.at[wid])


@functools.cache
def _gather_call():
    return functools.partial(
        pl.kernel,
        mesh=plsc.VectorSubcoreMesh(core_axis_name="c", subcore_axis_name="s",
                                    num_cores=_NC, num_subcores=_NS),
        out_type=jax.ShapeDtypeStruct((_NW, _PW, 8), jnp.float32),
        scratch_types=[
            pltpu.VMEM((_NT, _CH), jnp.int32),
            pltpu.VMEM((_PW, 8), jnp.float32),
            pltpu.SemaphoreType.DMA,
        ],
        compiler_params=pltpu.CompilerParams(use_tc_tiling_on_sc=False),
    )(_gather_body)


def _placement_matrices():
    P = np.zeros((8 * _S, 66), dtype=np.float32)
    for d in range(3):
        for t in range(1, _S):
            P[8 * t + d, 3 * (t - 1) + d] = 1.0
        for t in range(_S):
            P[8 * t + 3 + d, 30 + 3 * t + d] = 1.0
    R = np.zeros((3, 66), dtype=np.float32)
    Q3 = np.zeros((3, 6), dtype=np.float32)
    for d in range(3):
        for t in range(_K):
            R[d, 3 * t + d] = -1.0
        Q3[d, d] = 1.0
    return jnp.asarray(P), jnp.asarray(R), jnp.asarray(Q3)


def _mlp_body(g_ref, pos_ref, init_ref, W1_ref, W2_ref, W3_ref, b1_ref,
              b2_ref, b3_ref, P_ref, R_ref, Q3_ref, out_ref):
    f32 = jnp.float32
    hi = lax.Precision.HIGHEST
    W32 = jnp.dot(W3_ref[...], W2_ref[...], precision=hi,
                  preferred_element_type=f32)
    Weff = jnp.dot(W32, W1_ref[...], precision=hi,
                   preferred_element_type=f32)
    A = lax.dot_general(P_ref[...], Weff, (((1,), (1,)), ((), ())),
                        precision=hi, preferred_element_type=f32)
    C = lax.dot_general(R_ref[...], Weff, (((1,), (1,)), ((), ())),
                        precision=hi,
                        preferred_element_type=f32) + Q3_ref[...]
    beff = (lax.dot_general(b2_ref[...], W3_ref[...], (((1,), (1,)), ((), ())),
                            precision=hi, preferred_element_type=f32)
            + lax.dot_general(b1_ref[...], W32, (((1,), (1,)), ((), ())),
                              precision=hi, preferred_element_type=f32)
            + b3_ref[...])
    db = lax.dot_general(init_ref[0], Weff[:, 63:66],
                         (((1,), (1,)), ((), ())),
                         precision=hi, preferred_element_type=f32) + beff
    out_ref[0] = (jnp.dot(g_ref[0], A, precision=hi,
                          preferred_element_type=f32)
                  + jnp.dot(pos_ref[0], C, precision=hi,
                            preferred_element_type=f32) + db)


def _mlp_call(G, position, init3, W1, W2, W3, b1r, b2r, b3r, P, R, Q3):
    full = lambda shape: pl.BlockSpec(shape, lambda b, r: tuple(0 for _ in shape))
    return pl.pallas_call(
        _mlp_body,
        grid=(_B, _N // _RC),
        in_specs=[
            pl.BlockSpec((1, _RC, 8 * _S), lambda b, r: (b, r, 0)),
            pl.BlockSpec((1, _RC, 3), lambda b, r: (b, r, 0)),
            pl.BlockSpec((1, 1, 3), lambda b, r: (b, 0, 0)),
            full(W1.shape), full(W2.shape), full(W3.shape),
            full(b1r.shape), full(b2r.shape), full(b3r.shape),
            full(P.shape), full(R.shape), full(Q3.shape),
        ],
        out_specs=pl.BlockSpec((1, _RC, 6), lambda b, r: (b, r, 0)),
        out_shape=jax.ShapeDtypeStruct((_B, _N, 6), jnp.float32),
    )(G, position, init3, W1, W2, W3, b1r, b2r, b3r, P, R, Q3)


@jax.jit
def kernel(position, velocity, init, W1, b1, W2, b2, W3, b3):
    posT = jnp.swapaxes(position, 1, 2)
    idx = _topk_call(position, posT)
    idx3 = idx.reshape(_NW, _NT, _CH)

    table = jnp.concatenate(
        [position, velocity, jnp.zeros((_B, _N, 2), jnp.float32)], axis=-1
    ).reshape(_B * _N, 8)
    G = _gather_call()(table, idx3).reshape(_B, _N, 8 * _S)

    P, R, Q3 = _placement_matrices()
    return _mlp_call(G, position, init.reshape(_B, 1, 3), W1, W2, W3,
                     b1.reshape(1, -1), b2.reshape(1, -1), b3.reshape(1, -1),
                     P, R, Q3)


# speedup vs baseline: 29.7608x; 1.0515x over previous
import functools

import numpy as np
import jax
import jax.numpy as jnp
from jax import lax
from jax.experimental import pallas as pl
from jax.experimental.pallas import tpu as pltpu
from jax.experimental.pallas import tpu_sc as plsc

_B, _N, _K = 4, 4096, 10
_S = _K + 1
_R = 512
_RC = 1024

_NC, _NS = 2, 16
_NW = _NC * _NS
_TOT = _B * _N * _S
_PW = _TOT // _NW
_CH = 128
_NT = _PW // _CH
_GRP = 11


def _topk_body(pos_ref, posT_ref, idx_ref):
    b = pl.program_id(0)
    pos_r = pos_ref[0]
    posT = posT_ref[0]
    p2r = jnp.sum(pos_r * pos_r, axis=1, keepdims=True)
    p2c = jnp.sum(posT * posT, axis=0, keepdims=True)
    cross = jnp.dot(pos_r, posT, preferred_element_type=jnp.float32)
    d2 = p2r + p2c - 2.0 * cross
    colf = lax.broadcasted_iota(jnp.int32, d2.shape, 1).astype(jnp.float32)
    bigf = jnp.float32(_N)
    slots = []
    for _ in range(_S):
        m = jnp.min(d2, axis=1, keepdims=True)
        eq = d2 == m
        idx_t = jnp.min(jnp.where(eq, colf, bigf), axis=1, keepdims=True)
        slots.append(idx_t.astype(jnp.int32))
        d2 = jnp.where(eq, jnp.inf, d2)
    idx = jnp.concatenate(slots, axis=1)
    idx_ref[0] = idx + b * _N


def _topk_call(position, posT):
    return pl.pallas_call(
        _topk_body,
        grid=(_B, _N // _R),
        in_specs=[
            pl.BlockSpec((1, _R, 3), lambda b, r: (b, r, 0)),
            pl.BlockSpec((1, 3, _N), lambda b, r: (b, 0, 0)),
        ],
        out_specs=pl.BlockSpec((1, _R, _S), lambda b, r: (b, r, 0)),
        out_shape=jax.ShapeDtypeStruct((_B, _N, _S), jnp.int32),
    )(position, posT)


def _gather_body(table_ref, idx_ref, out_ref, idx_v, rows_v, sem):
    wid = lax.axis_index("s") * _NC + lax.axis_index("c")
    pltpu.sync_copy(idx_ref.at[wid], idx_v)

    def grp(gi, carry):
        handles = []
        for j in range(_GRP):
            tr = gi * _GRP + j
            cp = pltpu.async_copy(
                table_ref.at[idx_v.at[tr]],
                rows_v.at[pl.ds(tr * _CH, _CH)],
                sem,
            )
            handles.append(cp)
        for cp in handles:
            cp.wait()
        return carry

    lax.fori_loop(0, _NT // _GRP, grp, 0)
    pltpu.sync_copy(rows_v, out_ref.at[wid])


@functools.cache
def _gather_call():
    return functools.partial(
        pl.kernel,
        mesh=plsc.VectorSubcoreMesh(core_axis_name="c", subcore_axis_name="s",
                                    num_cores=_NC, num_subcores=_NS),
        out_type=jax.ShapeDtypeStruct((_NW, _PW, 8), jnp.float32),
        scratch_types=[
            pltpu.VMEM((_NT, _CH), jnp.int32),
            pltpu.VMEM((_PW, 8), jnp.float32),
            pltpu.SemaphoreType.DMA,
        ],
        compiler_params=pltpu.CompilerParams(use_tc_tiling_on_sc=False),
    )(_gather_body)


def _placement_matrices():
    P = np.zeros((8 * _S, 66), dtype=np.float32)
    for d in range(3):
        for t in range(1, _S):
            P[8 * t + d, 3 * (t - 1) + d] = 1.0
        for t in range(_S):
            P[8 * t + 3 + d, 30 + 3 * t + d] = 1.0
    R = np.zeros((3, 66), dtype=np.float32)
    Q3 = np.zeros((3, 6), dtype=np.float32)
    for d in range(3):
        for t in range(_K):
            R[d, 3 * t + d] = -1.0
        Q3[d, d] = 1.0
    return jnp.asarray(P), jnp.asarray(R), jnp.asarray(Q3)


def _mlp_body(g_ref, pos_ref, init_ref, W1_ref, W2_ref, W3_ref, b1_ref,
              b2_ref, b3_ref, P_ref, R_ref, Q3_ref, out_ref):
    f32 = jnp.float32
    hi = lax.Precision.HIGHEST
    W32 = jnp.dot(W3_ref[...], W2_ref[...], precision=hi,
                  preferred_element_type=f32)
    Weff = jnp.dot(W32, W1_ref[...], precision=hi,
                   preferred_element_type=f32)
    A = lax.dot_general(P_ref[...], Weff, (((1,), (1,)), ((), ())),
                        precision=hi, preferred_element_type=f32)
    C = lax.dot_general(R_ref[...], Weff, (((1,), (1,)), ((), ())),
                        precision=hi,
                        preferred_element_type=f32) + Q3_ref[...]
    beff = (lax.dot_general(b2_ref[...], W3_ref[...], (((1,), (1,)), ((), ())),
                            precision=hi, preferred_element_type=f32)
            + lax.dot_general(b1_ref[...], W32, (((1,), (1,)), ((), ())),
                              precision=hi, preferred_element_type=f32)
            + b3_ref[...])
    db = lax.dot_general(init_ref[0], Weff[:, 63:66],
                         (((1,), (1,)), ((), ())),
                         precision=hi, preferred_element_type=f32) + beff
    out_ref[0] = (jnp.dot(g_ref[0], A, precision=hi,
                          preferred_element_type=f32)
                  + jnp.dot(pos_ref[0], C, precision=hi,
                            preferred_element_type=f32) + db)


def _mlp_call(G, position, init3, W1, W2, W3, b1r, b2r, b3r, P, R, Q3):
    full = lambda shape: pl.BlockSpec(shape, lambda b, r: tuple(0 for _ in shape))
    return pl.pallas_call(
        _mlp_body,
        grid=(_B, _N // _RC),
        in_specs=[
            pl.BlockSpec((1, _RC, 8 * _S), lambda b, r: (b, r, 0)),
            pl.BlockSpec((1, _RC, 3), lambda b, r: (b, r, 0)),
            pl.BlockSpec((1, 1, 3), lambda b, r: (b, 0, 0)),
            full(W1.shape), full(W2.shape), full(W3.shape),
            full(b1r.shape), full(b2r.shape), full(b3r.shape),
            full(P.shape), full(R.shape), full(Q3.shape),
        ],
        out_specs=pl.BlockSpec((1, _RC, 6), lambda b, r: (b, r, 0)),
        out_shape=jax.ShapeDtypeStruct((_B, _N, 6), jnp.float32),
    )(G, position, init3, W1, W2, W3, b1r, b2r, b3r, P, R, Q3)


@jax.jit
def kernel(position, velocity, init, W1, b1, W2, b2, W3, b3):
    posT = jnp.swapaxes(position, 1, 2)
    idx = _topk_call(position, posT)
    idx3 = idx.reshape(_NW, _NT, _CH)

    table = jnp.concatenate(
        [position, velocity, jnp.zeros((_B, _N, 2), jnp.float32)], axis=-1
    ).reshape(_B * _N, 8)
    G = _gather_call()(table, idx3).reshape(_B, _N, 8 * _S)

    P, R, Q3 = _placement_matrices()
    return _mlp_call(G, position, init.reshape(_B, 1, 3), W1, W2, W3,
                     b1.reshape(1, -1), b2.reshape(1, -1), b3.reshape(1, -1),
                     P, R, Q3)
